# Initial kernel scaffold; baseline (speedup 1.0000x reference)
import functools

import jax
import jax.numpy as jnp
from jax import lax
from jax.experimental import pallas as pl
from jax.experimental.pallas import tpu as pltpu
from jax.experimental.pallas import tpu_sc as plsc

N = 10000
E = 320000
DIN = 128
H = 32
DOUT = 128

NC = 2
NS = 16
NW = NC * NS
CHUNK = 128
NCHUNK = 80
EPW = NCHUNK * CHUNK
EP = NW * EPW
NPAD = 10240
RPT = NPAD // NS

_mesh = plsc.VectorSubcoreMesh(core_axis_name="c", subcore_axis_name="s")


def _deg_body(dsts_hbm, ones_hbm, zeros_hbm, out_hbm, didx, dcur, ones_v, acc):
    c = lax.axis_index("c")
    s = lax.axis_index("s")
    wid = s * NC + c
    pltpu.sync_copy(zeros_hbm.at[pl.ds(s * RPT, RPT)],
                    acc.at[pl.ds(s * RPT, RPT)])
    pltpu.sync_copy(ones_hbm, ones_v)
    pltpu.sync_copy(dsts_hbm.at[wid], didx)
    plsc.subcore_barrier()

    def body(j, carry):
        pltpu.sync_copy(didx.at[j], dcur)
        pltpu.sync_copy(ones_v, acc.at[dcur], add=True)
        return carry

    lax.fori_loop(0, NCHUNK, body, 0)
    plsc.subcore_barrier()
    pltpu.sync_copy(acc.at[pl.ds(s * RPT, RPT)],
                    out_hbm.at[c, pl.ds(s * RPT, RPT)])


_deg_call = pl.kernel(
    _deg_body,
    out_type=jax.ShapeDtypeStruct((NC, NPAD, 8), jnp.float32),
    mesh=_mesh,
    scratch_types=[
        pltpu.VMEM((NCHUNK, CHUNK), jnp.int32),
        pltpu.VMEM((CHUNK,), jnp.int32),
        pltpu.VMEM((CHUNK, 8), jnp.float32),
        pltpu.VMEM_SHARED((NPAD, 8), jnp.float32),
    ],
)


def _prop_body(g_hbm, srcs_hbm, dsts_hbm, zeros_hbm, out_hbm,
               sidx, didx, scur, dcur, rows, acc, sem):
    c = lax.axis_index("c")
    s = lax.axis_index("s")
    wid = s * NC + c
    pltpu.sync_copy(zeros_hbm.at[pl.ds(s * RPT, RPT)],
                    acc.at[pl.ds(s * RPT, RPT)])
    pltpu.sync_copy(srcs_hbm.at[wid], sidx)
    pltpu.sync_copy(dsts_hbm.at[wid], didx)
    plsc.subcore_barrier()

    def body(j, carry):
        pltpu.sync_copy(sidx.at[j], scur)
        pltpu.sync_copy(didx.at[j], dcur)
        pltpu.async_copy(g_hbm.at[scur], rows, sem).wait()
        pltpu.sync_copy(rows, acc.at[dcur], add=True)
        return carry

    lax.fori_loop(0, NCHUNK, body, 0)
    plsc.subcore_barrier()
    pltpu.sync_copy(acc.at[pl.ds(s * RPT, RPT)],
                    out_hbm.at[c, pl.ds(s * RPT, RPT)])


_prop_call = pl.kernel(
    _prop_body,
    out_type=jax.ShapeDtypeStruct((NC, NPAD, H), jnp.float32),
    mesh=_mesh,
    scratch_types=[
        pltpu.VMEM((NCHUNK, CHUNK), jnp.int32),
        pltpu.VMEM((NCHUNK, CHUNK), jnp.int32),
        pltpu.VMEM((CHUNK,), jnp.int32),
        pltpu.VMEM((CHUNK,), jnp.int32),
        pltpu.VMEM((CHUNK, H), jnp.float32),
        pltpu.VMEM_SHARED((NPAD, H), jnp.float32),
        pltpu.SemaphoreType.DMA,
    ],
)

_BM = 1000
_GRID = N // _BM


def _mm0_body(x_ref, w_ref, b_ref, p0_ref, p1_ref, h_ref, g_ref):
    h = jnp.dot(x_ref[...], w_ref[...],
                preferred_element_type=jnp.float32) + b_ref[...]
    deg = 1.0 + p0_ref[:, :1] + p1_ref[:, :1]
    dis = lax.rsqrt(deg)
    h_ref[...] = h
    g_ref[...] = h * dis


def _round_body(a0_ref, a1_ref, h_ref, p0_ref, p1_ref, w_ref, b_ref,
                ho_ref, go_ref):
    deg = 1.0 + p0_ref[:, :1] + p1_ref[:, :1]
    dis = lax.rsqrt(deg)
    prop = dis * (a0_ref[...] + a1_ref[...]) + h_ref[...] / deg
    h = jnp.maximum(
        jnp.dot(prop, w_ref[...], preferred_element_type=jnp.float32)
        + b_ref[...], 0.0)
    ho_ref[...] = h
    go_ref[...] = h * dis


def _final_body(a0_ref, a1_ref, h_ref, p0_ref, p1_ref, w7_ref, b7_ref,
                w8_ref, b8_ref, o_ref):
    deg = 1.0 + p0_ref[:, :1] + p1_ref[:, :1]
    dis = lax.rsqrt(deg)
    prop = dis * (a0_ref[...] + a1_ref[...]) + h_ref[...] / deg
    h7 = jnp.maximum(
        jnp.dot(prop, w7_ref[...], preferred_element_type=jnp.float32)
        + b7_ref[...], 0.0)
    o_ref[...] = jnp.dot(h7, w8_ref[...],
                         preferred_element_type=jnp.float32) + b8_ref[...]


def _row_spec(width):
    return pl.BlockSpec((_BM, width), lambda i: (i, 0))


def _full_spec(shape):
    return pl.BlockSpec(shape, lambda i: (0,) * len(shape))


_params = pltpu.CompilerParams(dimension_semantics=("arbitrary",))

_mm0 = pl.pallas_call(
    _mm0_body,
    grid=(_GRID,),
    in_specs=[_row_spec(DIN), _full_spec((DIN, H)), _full_spec((1, H)),
              _row_spec(8), _row_spec(8)],
    out_specs=[_row_spec(H), _row_spec(H)],
    out_shape=[jax.ShapeDtypeStruct((N, H), jnp.float32),
               jax.ShapeDtypeStruct((N, H), jnp.float32)],
    compiler_params=_params,
)

_round = pl.pallas_call(
    _round_body,
    grid=(_GRID,),
    in_specs=[_row_spec(H), _row_spec(H), _row_spec(H),
              _row_spec(8), _row_spec(8),
              _full_spec((H, H)), _full_spec((1, H))],
    out_specs=[_row_spec(H), _row_spec(H)],
    out_shape=[jax.ShapeDtypeStruct((N, H), jnp.float32),
               jax.ShapeDtypeStruct((N, H), jnp.float32)],
    compiler_params=_params,
)

_final = pl.pallas_call(
    _final_body,
    grid=(_GRID,),
    in_specs=[_row_spec(H), _row_spec(H), _row_spec(H),
              _row_spec(8), _row_spec(8),
              _full_spec((H, H)), _full_spec((1, H)),
              _full_spec((H, DOUT)), _full_spec((1, DOUT))],
    out_specs=_row_spec(DOUT),
    out_shape=jax.ShapeDtypeStruct((N, DOUT), jnp.float32),
    compiler_params=_params,
)


def kernel(x, edge_index, W0, b0, W1, b1, W2, b2, W3, b3, W4, b4, W5, b5,
           W6, b6, W7, b7, W8, b8):
    src = edge_index[0]
    dst = edge_index[1]
    pad = EP - E
    srcp = jnp.concatenate(
        [src, jnp.zeros((pad,), jnp.int32)]).reshape(NW, NCHUNK, CHUNK)
    dstp = jnp.concatenate(
        [dst, jnp.full((pad,), N, jnp.int32)]).reshape(NW, NCHUNK, CHUNK)
    zeros_h = jnp.zeros((NPAD, H), jnp.float32)
    zeros_8 = jnp.zeros((NPAD, 8), jnp.float32)
    ones_8 = jnp.ones((CHUNK, 8), jnp.float32)

    degp = _deg_call(dstp, ones_8, zeros_8)
    p0 = degp[0]
    p1 = degp[1]

    h, g = _mm0(x, W0, b0.reshape(1, H), p0, p1)
    for Wi, bi in ((W1, b1), (W2, b2), (W3, b3), (W4, b4), (W5, b5),
                   (W6, b6)):
        acc = _prop_call(g, srcp, dstp, zeros_h)
        h, g = _round(acc[0], acc[1], h, p0, p1, Wi, bi.reshape(1, H))
    acc = _prop_call(g, srcp, dstp, zeros_h)
    return _final(acc[0], acc[1], h, p0, p1, W7, b7.reshape(1, H),
                  W8, b8.reshape(1, DOUT))


# speedup vs baseline: 13.6003x; 13.6003x over previous
import functools

import jax
import jax.numpy as jnp
from jax import lax
from jax.experimental import pallas as pl
from jax.experimental.pallas import tpu as pltpu
from jax.experimental.pallas import tpu_sc as plsc

N = 10000
E = 320000
DIN = 128
H = 32
DOUT = 128

NC = 2
NS = 16
NW = NC * NS
CHUNK = 128
NCHUNK = 80
EPW = NCHUNK * CHUNK
EP = NW * EPW
NPAD = 10240
RPT = NPAD // NS

_mesh = plsc.VectorSubcoreMesh(core_axis_name="c", subcore_axis_name="s")
_sc_params = pltpu.CompilerParams(use_tc_tiling_on_sc=False)


def _deg_body(dsts_hbm, ones_hbm, zeros_hbm, out_hbm, didx, ones_v, acc):
    c = lax.axis_index("c")
    s = lax.axis_index("s")
    wid = s * NC + c
    pltpu.sync_copy(zeros_hbm.at[pl.ds(s * RPT, RPT)],
                    acc.at[pl.ds(s * RPT, RPT)])
    pltpu.sync_copy(ones_hbm, ones_v)
    pltpu.sync_copy(dsts_hbm.at[wid], didx)
    plsc.subcore_barrier()

    def body(j, carry):
        pltpu.sync_copy(ones_v, acc.at[didx.at[j]], add=True)
        return carry

    lax.fori_loop(0, NCHUNK, body, 0)
    plsc.subcore_barrier()
    pltpu.sync_copy(acc.at[pl.ds(s * RPT, RPT)],
                    out_hbm.at[c, pl.ds(s * RPT, RPT)])


_deg_call = pl.kernel(
    _deg_body,
    out_type=jax.ShapeDtypeStruct((NC, NPAD, 8), jnp.float32),
    mesh=_mesh,
    scratch_types=[
        pltpu.VMEM((NCHUNK, CHUNK), jnp.int32),
        pltpu.VMEM((CHUNK, 8), jnp.float32),
        pltpu.VMEM_SHARED((NPAD, 8), jnp.float32),
    ],
    compiler_params=_sc_params,
)


def _prop_body(g_hbm, srcs_hbm, dsts_hbm, zeros_hbm, out_hbm,
               sidx, didx, rows, acc, sem):
    c = lax.axis_index("c")
    s = lax.axis_index("s")
    wid = s * NC + c
    pltpu.sync_copy(zeros_hbm.at[pl.ds(s * RPT, RPT)],
                    acc.at[pl.ds(s * RPT, RPT)])
    pltpu.sync_copy(srcs_hbm.at[wid], sidx)
    pltpu.sync_copy(dsts_hbm.at[wid], didx)
    plsc.subcore_barrier()

    def body(j, carry):
        pltpu.async_copy(g_hbm.at[sidx.at[j]], rows, sem).wait()
        pltpu.sync_copy(rows, acc.at[didx.at[j]], add=True)
        return carry

    lax.fori_loop(0, NCHUNK, body, 0)
    plsc.subcore_barrier()
    pltpu.sync_copy(acc.at[pl.ds(s * RPT, RPT)],
                    out_hbm.at[c, pl.ds(s * RPT, RPT)])


_prop_call = pl.kernel(
    _prop_body,
    out_type=jax.ShapeDtypeStruct((NC, NPAD, H), jnp.float32),
    mesh=_mesh,
    scratch_types=[
        pltpu.VMEM((NCHUNK, CHUNK), jnp.int32),
        pltpu.VMEM((NCHUNK, CHUNK), jnp.int32),
        pltpu.VMEM((CHUNK, H), jnp.float32),
        pltpu.VMEM_SHARED((NPAD, H), jnp.float32),
        pltpu.SemaphoreType.DMA,
    ],
    compiler_params=_sc_params,
)

_BM = 1000
_GRID = N // _BM


def _mm0_body(x_ref, w_ref, b_ref, p0_ref, p1_ref, h_ref, g_ref):
    h = jnp.dot(x_ref[...], w_ref[...],
                preferred_element_type=jnp.float32) + b_ref[...]
    deg = 1.0 + p0_ref[:, :1] + p1_ref[:, :1]
    dis = lax.rsqrt(deg)
    h_ref[...] = h
    g_ref[...] = h * dis


def _round_body(a0_ref, a1_ref, h_ref, p0_ref, p1_ref, w_ref, b_ref,
                ho_ref, go_ref):
    deg = 1.0 + p0_ref[:, :1] + p1_ref[:, :1]
    dis = lax.rsqrt(deg)
    prop = dis * (a0_ref[...] + a1_ref[...]) + h_ref[...] / deg
    h = jnp.maximum(
        jnp.dot(prop, w_ref[...], preferred_element_type=jnp.float32)
        + b_ref[...], 0.0)
    ho_ref[...] = h
    go_ref[...] = h * dis


def _final_body(a0_ref, a1_ref, h_ref, p0_ref, p1_ref, w7_ref, b7_ref,
                w8_ref, b8_ref, o_ref):
    deg = 1.0 + p0_ref[:, :1] + p1_ref[:, :1]
    dis = lax.rsqrt(deg)
    prop = dis * (a0_ref[...] + a1_ref[...]) + h_ref[...] / deg
    h7 = jnp.maximum(
        jnp.dot(prop, w7_ref[...], preferred_element_type=jnp.float32)
        + b7_ref[...], 0.0)
    o_ref[...] = jnp.dot(h7, w8_ref[...],
                         preferred_element_type=jnp.float32) + b8_ref[...]


def _row_spec(width):
    return pl.BlockSpec((_BM, width), lambda i: (i, 0))


def _full_spec(shape):
    return pl.BlockSpec(shape, lambda i: (0,) * len(shape))


_params = pltpu.CompilerParams(dimension_semantics=("arbitrary",))

_mm0 = pl.pallas_call(
    _mm0_body,
    grid=(_GRID,),
    in_specs=[_row_spec(DIN), _full_spec((DIN, H)), _full_spec((1, H)),
              _row_spec(8), _row_spec(8)],
    out_specs=[_row_spec(H), _row_spec(H)],
    out_shape=[jax.ShapeDtypeStruct((N, H), jnp.float32),
               jax.ShapeDtypeStruct((N, H), jnp.float32)],
    compiler_params=_params,
)

_round = pl.pallas_call(
    _round_body,
    grid=(_GRID,),
    in_specs=[_row_spec(H), _row_spec(H), _row_spec(H),
              _row_spec(8), _row_spec(8),
              _full_spec((H, H)), _full_spec((1, H))],
    out_specs=[_row_spec(H), _row_spec(H)],
    out_shape=[jax.ShapeDtypeStruct((N, H), jnp.float32),
               jax.ShapeDtypeStruct((N, H), jnp.float32)],
    compiler_params=_params,
)

_final = pl.pallas_call(
    _final_body,
    grid=(_GRID,),
    in_specs=[_row_spec(H), _row_spec(H), _row_spec(H),
              _row_spec(8), _row_spec(8),
              _full_spec((H, H)), _full_spec((1, H)),
              _full_spec((H, DOUT)), _full_spec((1, DOUT))],
    out_specs=_row_spec(DOUT),
    out_shape=jax.ShapeDtypeStruct((N, DOUT), jnp.float32),
    compiler_params=_params,
)


def kernel(x, edge_index, W0, b0, W1, b1, W2, b2, W3, b3, W4, b4, W5, b5,
           W6, b6, W7, b7, W8, b8):
    src = edge_index[0]
    dst = edge_index[1]
    pad = EP - E
    srcp = jnp.concatenate(
        [src, jnp.zeros((pad,), jnp.int32)]).reshape(NW, NCHUNK, CHUNK)
    dstp = jnp.concatenate(
        [dst, jnp.full((pad,), N, jnp.int32)]).reshape(NW, NCHUNK, CHUNK)
    zeros_h = jnp.zeros((NPAD, H), jnp.float32)
    zeros_8 = jnp.zeros((NPAD, 8), jnp.float32)
    ones_8 = jnp.ones((CHUNK, 8), jnp.float32)

    degp = _deg_call(dstp, ones_8, zeros_8)
    p0 = degp[0]
    p1 = degp[1]

    h, g = _mm0(x, W0, b0.reshape(1, H), p0, p1)
    for Wi, bi in ((W1, b1), (W2, b2), (W3, b3), (W4, b4), (W5, b5),
                   (W6, b6)):
        acc = _prop_call(g, srcp, dstp, zeros_h)
        h, g = _round(acc[0], acc[1], h, p0, p1, Wi, bi.reshape(1, H))
    acc = _prop_call(g, srcp, dstp, zeros_h)
    return _final(acc[0], acc[1], h, p0, p1, W7, b7.reshape(1, H),
                  W8, b8.reshape(1, DOUT))


# speedup vs baseline: 16.6501x; 1.2243x over previous
import functools

import jax
import jax.numpy as jnp
from jax import lax
from jax.experimental import pallas as pl
from jax.experimental.pallas import tpu as pltpu
from jax.experimental.pallas import tpu_sc as plsc

N = 10000
E = 320000
DIN = 128
H = 32
DOUT = 128

NC = 2
NS = 16
NW = NC * NS
CHUNK = 128
NCHUNK = 80
EPW = NCHUNK * CHUNK
EP = NW * EPW
NPAD = 10240
RPT = NPAD // NS

_mesh = plsc.VectorSubcoreMesh(core_axis_name="c", subcore_axis_name="s")
_sc_params = pltpu.CompilerParams(use_tc_tiling_on_sc=False)


def _deg_body(dsts_hbm, ones_hbm, zeros_hbm, out_hbm, didx, ones_v, acc, sem):
    c = lax.axis_index("c")
    s = lax.axis_index("s")
    wid = s * NC + c
    pltpu.sync_copy(zeros_hbm.at[pl.ds(s * RPT, RPT)],
                    acc.at[pl.ds(s * RPT, RPT)])
    pltpu.sync_copy(ones_hbm, ones_v)
    pltpu.sync_copy(dsts_hbm.at[wid], didx)
    plsc.subcore_barrier()

    def body(i, carry):
        for b in range(8):
            pltpu.async_copy(ones_v, acc.at[didx.at[8 * i + b]], sem,
                             add=True)
        for b in range(8):
            pltpu.make_async_copy(ones_v, acc.at[didx.at[0]], sem).wait()
        return carry

    lax.fori_loop(0, NCHUNK // 8, body, 0)
    plsc.subcore_barrier()
    pltpu.sync_copy(acc.at[pl.ds(s * RPT, RPT)],
                    out_hbm.at[c, pl.ds(s * RPT, RPT)])


_deg_call = pl.kernel(
    _deg_body,
    out_type=jax.ShapeDtypeStruct((NC, NPAD, 8), jnp.float32),
    mesh=_mesh,
    scratch_types=[
        pltpu.VMEM((NCHUNK, CHUNK), jnp.int32),
        pltpu.VMEM((CHUNK, 8), jnp.float32),
        pltpu.VMEM_SHARED((NPAD, 8), jnp.float32),
        pltpu.SemaphoreType.DMA,
    ],
    compiler_params=_sc_params,
)


def _prop_body(g_hbm, srcs_hbm, dsts_hbm, zeros_hbm, out_hbm,
               sidx, didx, rows, acc, gsa, gsb, ssa, ssb):
    c = lax.axis_index("c")
    s = lax.axis_index("s")
    wid = s * NC + c
    pltpu.sync_copy(zeros_hbm.at[pl.ds(s * RPT, RPT)],
                    acc.at[pl.ds(s * RPT, RPT)])
    pltpu.sync_copy(srcs_hbm.at[wid], sidx)
    pltpu.sync_copy(dsts_hbm.at[wid], didx)
    plsc.subcore_barrier()

    def _wait_g(sem, b):
        pltpu.make_async_copy(g_hbm.at[sidx.at[0]], rows.at[b], sem).wait()

    def _wait_s(sem, b):
        pltpu.make_async_copy(rows.at[b], acc.at[didx.at[0]], sem).wait()

    def body(i, carry):
        base = 8 * i

        @pl.when(i > 0)
        def _():
            for b in range(4):
                _wait_s(ssa, b)

        for b in range(4):
            pltpu.async_copy(g_hbm.at[sidx.at[base + b]], rows.at[b], gsa)
        for b in range(4):
            _wait_g(gsa, b)

        @pl.when(i > 0)
        def _():
            for b in range(4):
                _wait_s(ssb, 4 + b)

        for b in range(4):
            pltpu.async_copy(g_hbm.at[sidx.at[base + 4 + b]],
                             rows.at[4 + b], gsb)
        for b in range(4):
            pltpu.async_copy(rows.at[b], acc.at[didx.at[base + b]], ssa,
                             add=True)
        for b in range(4):
            _wait_g(gsb, 4 + b)
        for b in range(4):
            pltpu.async_copy(rows.at[4 + b], acc.at[didx.at[base + 4 + b]],
                             ssb, add=True)
        return carry

    lax.fori_loop(0, NCHUNK // 8, body, 0)
    for b in range(4):
        _wait_s(ssa, b)
    for b in range(4):
        _wait_s(ssb, 4 + b)
    plsc.subcore_barrier()
    pltpu.sync_copy(acc.at[pl.ds(s * RPT, RPT)],
                    out_hbm.at[c, pl.ds(s * RPT, RPT)])


_prop_call = pl.kernel(
    _prop_body,
    out_type=jax.ShapeDtypeStruct((NC, NPAD, H), jnp.float32),
    mesh=_mesh,
    scratch_types=[
        pltpu.VMEM((NCHUNK, CHUNK), jnp.int32),
        pltpu.VMEM((NCHUNK, CHUNK), jnp.int32),
        pltpu.VMEM((8, CHUNK, H), jnp.float32),
        pltpu.VMEM_SHARED((NPAD, H), jnp.float32),
        pltpu.SemaphoreType.DMA,
        pltpu.SemaphoreType.DMA,
        pltpu.SemaphoreType.DMA,
        pltpu.SemaphoreType.DMA,
    ],
    compiler_params=_sc_params,
)

_BM = 1000
_GRID = N // _BM


def _mm0_body(x_ref, w_ref, b_ref, p0_ref, p1_ref, h_ref, g_ref):
    h = jnp.dot(x_ref[...], w_ref[...],
                preferred_element_type=jnp.float32) + b_ref[...]
    deg = 1.0 + p0_ref[:, :1] + p1_ref[:, :1]
    dis = lax.rsqrt(deg)
    h_ref[...] = h
    g_ref[...] = h * dis


def _round_body(a0_ref, a1_ref, h_ref, p0_ref, p1_ref, w_ref, b_ref,
                ho_ref, go_ref):
    deg = 1.0 + p0_ref[:, :1] + p1_ref[:, :1]
    dis = lax.rsqrt(deg)
    prop = dis * (a0_ref[...] + a1_ref[...]) + h_ref[...] / deg
    h = jnp.maximum(
        jnp.dot(prop, w_ref[...], preferred_element_type=jnp.float32)
        + b_ref[...], 0.0)
    ho_ref[...] = h
    go_ref[...] = h * dis


def _final_body(a0_ref, a1_ref, h_ref, p0_ref, p1_ref, w7_ref, b7_ref,
                w8_ref, b8_ref, o_ref):
    deg = 1.0 + p0_ref[:, :1] + p1_ref[:, :1]
    dis = lax.rsqrt(deg)
    prop = dis * (a0_ref[...] + a1_ref[...]) + h_ref[...] / deg
    h7 = jnp.maximum(
        jnp.dot(prop, w7_ref[...], preferred_element_type=jnp.float32)
        + b7_ref[...], 0.0)
    o_ref[...] = jnp.dot(h7, w8_ref[...],
                         preferred_element_type=jnp.float32) + b8_ref[...]


def _row_spec(width):
    return pl.BlockSpec((_BM, width), lambda i: (i, 0))


def _full_spec(shape):
    return pl.BlockSpec(shape, lambda i: (0,) * len(shape))


_params = pltpu.CompilerParams(dimension_semantics=("arbitrary",))

_mm0 = pl.pallas_call(
    _mm0_body,
    grid=(_GRID,),
    in_specs=[_row_spec(DIN), _full_spec((DIN, H)), _full_spec((1, H)),
              _row_spec(8), _row_spec(8)],
    out_specs=[_row_spec(H), _row_spec(H)],
    out_shape=[jax.ShapeDtypeStruct((N, H), jnp.float32),
               jax.ShapeDtypeStruct((N, H), jnp.float32)],
    compiler_params=_params,
)

_round = pl.pallas_call(
    _round_body,
    grid=(_GRID,),
    in_specs=[_row_spec(H), _row_spec(H), _row_spec(H),
              _row_spec(8), _row_spec(8),
              _full_spec((H, H)), _full_spec((1, H))],
    out_specs=[_row_spec(H), _row_spec(H)],
    out_shape=[jax.ShapeDtypeStruct((N, H), jnp.float32),
               jax.ShapeDtypeStruct((N, H), jnp.float32)],
    compiler_params=_params,
)

_final = pl.pallas_call(
    _final_body,
    grid=(_GRID,),
    in_specs=[_row_spec(H), _row_spec(H), _row_spec(H),
              _row_spec(8), _row_spec(8),
              _full_spec((H, H)), _full_spec((1, H)),
              _full_spec((H, DOUT)), _full_spec((1, DOUT))],
    out_specs=_row_spec(DOUT),
    out_shape=jax.ShapeDtypeStruct((N, DOUT), jnp.float32),
    compiler_params=_params,
)


def kernel(x, edge_index, W0, b0, W1, b1, W2, b2, W3, b3, W4, b4, W5, b5,
           W6, b6, W7, b7, W8, b8):
    src = edge_index[0]
    dst = edge_index[1]
    pad = EP - E
    srcp = jnp.concatenate(
        [src, jnp.zeros((pad,), jnp.int32)]).reshape(NW, NCHUNK, CHUNK)
    dstp = jnp.concatenate(
        [dst, jnp.full((pad,), N, jnp.int32)]).reshape(NW, NCHUNK, CHUNK)
    zeros_h = jnp.zeros((NPAD, H), jnp.float32)
    zeros_8 = jnp.zeros((NPAD, 8), jnp.float32)
    ones_8 = jnp.ones((CHUNK, 8), jnp.float32)

    degp = _deg_call(dstp, ones_8, zeros_8)
    p0 = degp[0]
    p1 = degp[1]

    h, g = _mm0(x, W0, b0.reshape(1, H), p0, p1)
    for Wi, bi in ((W1, b1), (W2, b2), (W3, b3), (W4, b4), (W5, b5),
                   (W6, b6)):
        acc = _prop_call(g, srcp, dstp, zeros_h)
        h, g = _round(acc[0], acc[1], h, p0, p1, Wi, bi.reshape(1, H))
    acc = _prop_call(g, srcp, dstp, zeros_h)
    return _final(acc[0], acc[1], h, p0, p1, W7, b7.reshape(1, H),
                  W8, b8.reshape(1, DOUT))


# speedup vs baseline: 16.7445x; 1.0057x over previous
import functools

import jax
import jax.numpy as jnp
from jax import lax
from jax.experimental import pallas as pl
from jax.experimental.pallas import tpu as pltpu
from jax.experimental.pallas import tpu_sc as plsc

N = 10000
E = 320000
DIN = 128
H = 32
DOUT = 128

NC = 2
NS = 16
NW = NC * NS
CHUNK = 128
NCHUNK = 80
EPW = NCHUNK * CHUNK
EP = NW * EPW
NPAD = 10240
RPT = NPAD // NS

_mesh = plsc.VectorSubcoreMesh(core_axis_name="c", subcore_axis_name="s")
_sc_params = pltpu.CompilerParams(use_tc_tiling_on_sc=False)


def _deg_body(dsts_hbm, ones_hbm, zeros_hbm, out_hbm, didx, ones_v, acc, sem):
    c = lax.axis_index("c")
    s = lax.axis_index("s")
    wid = s * NC + c
    pltpu.sync_copy(zeros_hbm.at[pl.ds(s * RPT, RPT)],
                    acc.at[pl.ds(s * RPT, RPT)])
    pltpu.sync_copy(ones_hbm, ones_v)
    pltpu.sync_copy(dsts_hbm.at[wid], didx)
    plsc.subcore_barrier()

    def body(i, carry):
        for b in range(8):
            pltpu.async_copy(ones_v, acc.at[didx.at[8 * i + b]], sem,
                             add=True)
        for b in range(8):
            pltpu.make_async_copy(ones_v, acc.at[didx.at[0]], sem).wait()
        return carry

    lax.fori_loop(0, NCHUNK // 8, body, 0)
    plsc.subcore_barrier()
    pltpu.sync_copy(acc.at[pl.ds(s * RPT, RPT)],
                    out_hbm.at[c, pl.ds(s * RPT, RPT)])


_deg_call = pl.kernel(
    _deg_body,
    out_type=jax.ShapeDtypeStruct((NC, NPAD, 8), jnp.float32),
    mesh=_mesh,
    scratch_types=[
        pltpu.VMEM((NCHUNK, CHUNK), jnp.int32),
        pltpu.VMEM((CHUNK, 8), jnp.float32),
        pltpu.VMEM_SHARED((NPAD, 8), jnp.float32),
        pltpu.SemaphoreType.DMA,
    ],
    compiler_params=_sc_params,
)


def _prop_body(g_hbm, srcs_hbm, dsts_hbm, zeros_hbm, out_hbm,
               sidx, didx, rows, acc, gsa, gsb, ssa, ssb):
    c = lax.axis_index("c")
    s = lax.axis_index("s")
    wid = s * NC + c
    pltpu.sync_copy(zeros_hbm.at[pl.ds(s * RPT, RPT)],
                    acc.at[pl.ds(s * RPT, RPT)])
    pltpu.sync_copy(srcs_hbm.at[wid], sidx)
    pltpu.sync_copy(dsts_hbm.at[wid], didx)
    plsc.subcore_barrier()

    def _wait_g(sem, b):
        pltpu.make_async_copy(g_hbm.at[sidx.at[0]], rows.at[b], sem).wait()

    def _wait_s(sem, b):
        pltpu.make_async_copy(rows.at[b], acc.at[didx.at[0]], sem).wait()

    def body(i, carry):
        base = 8 * i

        @pl.when(i > 0)
        def _():
            for b in range(4):
                _wait_s(ssa, b)

        for b in range(4):
            pltpu.async_copy(g_hbm.at[sidx.at[base + b]], rows.at[b], gsa)
        for b in range(4):
            _wait_g(gsa, b)

        @pl.when(i > 0)
        def _():
            for b in range(4):
                _wait_s(ssb, 4 + b)

        for b in range(4):
            pltpu.async_copy(g_hbm.at[sidx.at[base + 4 + b]],
                             rows.at[4 + b], gsb)
        for b in range(4):
            pltpu.async_copy(rows.at[b], acc.at[didx.at[base + b]], ssa,
                             add=True)
        for b in range(4):
            _wait_g(gsb, 4 + b)
        for b in range(4):
            pltpu.async_copy(rows.at[4 + b], acc.at[didx.at[base + 4 + b]],
                             ssb, add=True)
        return carry

    lax.fori_loop(0, NCHUNK // 8, body, 0)
    for b in range(4):
        _wait_s(ssa, b)
    for b in range(4):
        _wait_s(ssb, 4 + b)
    plsc.subcore_barrier()
    pltpu.sync_copy(acc.at[pl.ds(s * RPT, RPT)],
                    out_hbm.at[c, pl.ds(s * RPT, RPT)])


_prop_call = pl.kernel(
    _prop_body,
    out_type=jax.ShapeDtypeStruct((NC, NPAD, H), jnp.float32),
    mesh=_mesh,
    scratch_types=[
        pltpu.VMEM((NCHUNK, CHUNK), jnp.int32),
        pltpu.VMEM((NCHUNK, CHUNK), jnp.int32),
        pltpu.VMEM((8, CHUNK, H), jnp.float32),
        pltpu.VMEM_SHARED((NPAD, H), jnp.float32),
        pltpu.SemaphoreType.DMA,
        pltpu.SemaphoreType.DMA,
        pltpu.SemaphoreType.DMA,
        pltpu.SemaphoreType.DMA,
    ],
    compiler_params=_sc_params,
)

_BM = 1000
_GRID = N // _BM


def _mm0_body(x_ref, w_ref, b_ref, p0_ref, p1_ref, h_ref, g_ref):
    h = jnp.dot(x_ref[...], w_ref[...],
                preferred_element_type=jnp.float32) + b_ref[...]
    deg = 1.0 + p0_ref[:, :1] + p1_ref[:, :1]
    dis = lax.rsqrt(deg)
    h_ref[...] = h
    g_ref[...] = h * dis


def _round_body(a0_ref, a1_ref, h_ref, p0_ref, p1_ref, w_ref, b_ref,
                ho_ref, go_ref):
    deg = 1.0 + p0_ref[:, :1] + p1_ref[:, :1]
    dis = lax.rsqrt(deg)
    prop = dis * (a0_ref[...] + a1_ref[...]) + h_ref[...] / deg
    h = jnp.maximum(
        jnp.dot(prop, w_ref[...], preferred_element_type=jnp.float32)
        + b_ref[...], 0.0)
    ho_ref[...] = h
    go_ref[...] = h * dis


def _final_body(a0_ref, a1_ref, h_ref, p0_ref, p1_ref, w7_ref, b7_ref,
                w8_ref, b8_ref, o_ref):
    deg = 1.0 + p0_ref[:, :1] + p1_ref[:, :1]
    dis = lax.rsqrt(deg)
    prop = dis * (a0_ref[...] + a1_ref[...]) + h_ref[...] / deg
    h7 = jnp.maximum(
        jnp.dot(prop, w7_ref[...], preferred_element_type=jnp.float32)
        + b7_ref[...], 0.0)
    o_ref[...] = jnp.dot(h7, w8_ref[...],
                         preferred_element_type=jnp.float32) + b8_ref[...]


def _row_spec(width):
    return pl.BlockSpec((_BM, width), lambda i: (i, 0))


def _full_spec(shape):
    return pl.BlockSpec(shape, lambda i: (0,) * len(shape))


_params = pltpu.CompilerParams(dimension_semantics=("arbitrary",))

_mm0 = pl.pallas_call(
    _mm0_body,
    grid=(_GRID,),
    in_specs=[_row_spec(DIN), _full_spec((DIN, H)), _full_spec((1, H)),
              _row_spec(8), _row_spec(8)],
    out_specs=[_row_spec(H), _row_spec(H)],
    out_shape=[jax.ShapeDtypeStruct((N, H), jnp.float32),
               jax.ShapeDtypeStruct((N, H), jnp.float32)],
    compiler_params=_params,
)

_round = pl.pallas_call(
    _round_body,
    grid=(_GRID,),
    in_specs=[_row_spec(H), _row_spec(H), _row_spec(H),
              _row_spec(8), _row_spec(8),
              _full_spec((H, H)), _full_spec((1, H))],
    out_specs=[_row_spec(H), _row_spec(H)],
    out_shape=[jax.ShapeDtypeStruct((N, H), jnp.float32),
               jax.ShapeDtypeStruct((N, H), jnp.float32)],
    compiler_params=_params,
)

_final = pl.pallas_call(
    _final_body,
    grid=(_GRID,),
    in_specs=[_row_spec(H), _row_spec(H), _row_spec(H),
              _row_spec(8), _row_spec(8),
              _full_spec((H, H)), _full_spec((1, H)),
              _full_spec((H, DOUT)), _full_spec((1, DOUT))],
    out_specs=_row_spec(DOUT),
    out_shape=jax.ShapeDtypeStruct((N, DOUT), jnp.float32),
    compiler_params=_params,
)


def kernel(x, edge_index, W0, b0, W1, b1, W2, b2, W3, b3, W4, b4, W5, b5,
           W6, b6, W7, b7, W8, b8):
    src = edge_index[0]
    dst = edge_index[1]
    pad = EP - E
    srcp = jnp.concatenate(
        [src, jnp.zeros((pad,), jnp.int32)]).reshape(NW, NCHUNK, CHUNK)
    pad_dst = N + jnp.arange(pad, dtype=jnp.int32) % (NPAD - N)
    dstp = jnp.concatenate([dst, pad_dst]).reshape(NW, NCHUNK, CHUNK)
    zeros_h = jnp.zeros((NPAD, H), jnp.float32)
    zeros_8 = jnp.zeros((NPAD, 8), jnp.float32)
    ones_8 = jnp.ones((CHUNK, 8), jnp.float32)

    degp = _deg_call(dstp, ones_8, zeros_8)
    p0 = degp[0]
    p1 = degp[1]

    h, g = _mm0(x, W0, b0.reshape(1, H), p0, p1)
    for Wi, bi in ((W1, b1), (W2, b2), (W3, b3), (W4, b4), (W5, b5),
                   (W6, b6)):
        acc = _prop_call(g, srcp, dstp, zeros_h)
        h, g = _round(acc[0], acc[1], h, p0, p1, Wi, bi.reshape(1, H))
    acc = _prop_call(g, srcp, dstp, zeros_h)
    return _final(acc[0], acc[1], h, p0, p1, W7, b7.reshape(1, H),
                  W8, b8.reshape(1, DOUT))


# speedup vs baseline: 33.5480x; 2.0035x over previous
import functools

import jax
import jax.numpy as jnp
from jax import lax
from jax.experimental import pallas as pl
from jax.experimental.pallas import tpu as pltpu
from jax.experimental.pallas import tpu_sc as plsc

N = 10000
E = 320000
DIN = 128
H = 32
DOUT = 128

NC = 2
NS = 16
NW = NC * NS
CHUNK = 128
NCHUNK = 80
EPW = NCHUNK * CHUNK
EP = NW * EPW
NPAD = 10240
RPT = NPAD // NS

_mesh = plsc.VectorSubcoreMesh(core_axis_name="c", subcore_axis_name="s")
_sc_params = pltpu.CompilerParams(use_tc_tiling_on_sc=False)


def _deg_body(dsts_hbm, ones_hbm, zeros_hbm, out_hbm, didx, ones_v, acc, sem):
    c = lax.axis_index("c")
    s = lax.axis_index("s")
    wid = s * NC + c
    pltpu.sync_copy(zeros_hbm.at[pl.ds(s * RPT, RPT)],
                    acc.at[pl.ds(s * RPT, RPT)])
    pltpu.sync_copy(ones_hbm, ones_v)
    pltpu.sync_copy(dsts_hbm.at[wid], didx)
    plsc.subcore_barrier()

    def body(i, carry):
        for b in range(8):
            pltpu.async_copy(ones_v, acc.at[didx.at[8 * i + b]], sem,
                             add=True)
        for b in range(8):
            pltpu.make_async_copy(ones_v, acc.at[didx.at[0]], sem).wait()
        return carry

    lax.fori_loop(0, NCHUNK // 8, body, 0)
    plsc.subcore_barrier()
    pltpu.sync_copy(acc.at[pl.ds(s * RPT, RPT)],
                    out_hbm.at[c, pl.ds(s * RPT, RPT)])


_deg_call = pl.kernel(
    _deg_body,
    out_type=jax.ShapeDtypeStruct((NC, NPAD, 8), jnp.float32),
    mesh=_mesh,
    scratch_types=[
        pltpu.VMEM((NCHUNK, CHUNK), jnp.int32),
        pltpu.VMEM((CHUNK, 8), jnp.float32),
        pltpu.VMEM_SHARED((NPAD, 8), jnp.float32),
        pltpu.SemaphoreType.DMA,
    ],
    compiler_params=_sc_params,
)


def _prop_body(g_hbm, srcs_hbm, dsts_hbm, zeros_hbm, out_hbm,
               sidx, didx, rows, acc, gsa, gsb, ssa, ssb):
    c = lax.axis_index("c")
    s = lax.axis_index("s")
    wid = s * NC + c
    pltpu.sync_copy(zeros_hbm.at[pl.ds(s * RPT, RPT)],
                    acc.at[pl.ds(s * RPT, RPT)])
    pltpu.sync_copy(srcs_hbm.at[wid], sidx)
    pltpu.sync_copy(dsts_hbm.at[wid], didx)
    plsc.subcore_barrier()

    def _wait_g(sem, b):
        pltpu.make_async_copy(g_hbm.at[sidx.at[0]], rows.at[b], sem).wait()

    def _wait_s(sem, b):
        pltpu.make_async_copy(rows.at[b], acc.at[didx.at[0]], sem).wait()

    def body(i, carry):
        base = 8 * i

        @pl.when(i > 0)
        def _():
            for b in range(4):
                _wait_s(ssa, b)

        for b in range(4):
            pltpu.async_copy(g_hbm.at[sidx.at[base + b]], rows.at[b], gsa)
        for b in range(4):
            _wait_g(gsa, b)

        @pl.when(i > 0)
        def _():
            for b in range(4):
                _wait_s(ssb, 4 + b)

        for b in range(4):
            pltpu.async_copy(g_hbm.at[sidx.at[base + 4 + b]],
                             rows.at[4 + b], gsb)
        for b in range(4):
            pltpu.async_copy(rows.at[b], acc.at[didx.at[base + b]], ssa,
                             add=True)
        for b in range(4):
            _wait_g(gsb, 4 + b)
        for b in range(4):
            pltpu.async_copy(rows.at[4 + b], acc.at[didx.at[base + 4 + b]],
                             ssb, add=True)
        return carry

    lax.fori_loop(0, NCHUNK // 8, body, 0)
    for b in range(4):
        _wait_s(ssa, b)
    for b in range(4):
        _wait_s(ssb, 4 + b)
    plsc.subcore_barrier()
    pltpu.sync_copy(acc.at[pl.ds(s * RPT, RPT)],
                    out_hbm.at[c, pl.ds(s * RPT, RPT)])


_prop_call = pl.kernel(
    _prop_body,
    out_type=jax.ShapeDtypeStruct((NC, NPAD, H), jnp.float32),
    mesh=_mesh,
    scratch_types=[
        pltpu.VMEM((NCHUNK, CHUNK), jnp.int32),
        pltpu.VMEM((NCHUNK, CHUNK), jnp.int32),
        pltpu.VMEM((8, CHUNK, H), jnp.float32),
        pltpu.VMEM_SHARED((NPAD, H), jnp.float32),
        pltpu.SemaphoreType.DMA,
        pltpu.SemaphoreType.DMA,
        pltpu.SemaphoreType.DMA,
        pltpu.SemaphoreType.DMA,
    ],
    compiler_params=_sc_params,
)

_BM = 1000
_GRID = N // _BM


def _mm0_body(x_ref, w_ref, b_ref, p0_ref, p1_ref, h_ref, g_ref):
    h = jnp.dot(x_ref[...], w_ref[...],
                preferred_element_type=jnp.float32) + b_ref[...]
    deg = 1.0 + p0_ref[:, :1] + p1_ref[:, :1]
    dis = lax.rsqrt(deg)
    h_ref[...] = h
    g_ref[...] = h * dis


def _round_body(a0_ref, a1_ref, h_ref, p0_ref, p1_ref, w_ref, b_ref,
                ho_ref, go_ref):
    deg = 1.0 + p0_ref[:, :1] + p1_ref[:, :1]
    dis = lax.rsqrt(deg)
    prop = dis * (a0_ref[...] + a1_ref[...]) + h_ref[...] / deg
    h = jnp.maximum(
        jnp.dot(prop, w_ref[...], preferred_element_type=jnp.float32)
        + b_ref[...], 0.0)
    ho_ref[...] = h
    go_ref[...] = h * dis


def _final_body(a0_ref, a1_ref, h_ref, p0_ref, p1_ref, w7_ref, b7_ref,
                w8_ref, b8_ref, o_ref):
    deg = 1.0 + p0_ref[:, :1] + p1_ref[:, :1]
    dis = lax.rsqrt(deg)
    prop = dis * (a0_ref[...] + a1_ref[...]) + h_ref[...] / deg
    h7 = jnp.maximum(
        jnp.dot(prop, w7_ref[...], preferred_element_type=jnp.float32)
        + b7_ref[...], 0.0)
    o_ref[...] = jnp.dot(h7, w8_ref[...],
                         preferred_element_type=jnp.float32) + b8_ref[...]


def _row_spec(width):
    return pl.BlockSpec((_BM, width), lambda i: (i, 0))


def _full_spec(shape):
    return pl.BlockSpec(shape, lambda i: (0,) * len(shape))


_params = pltpu.CompilerParams(dimension_semantics=("arbitrary",))

_mm0 = pl.pallas_call(
    _mm0_body,
    grid=(_GRID,),
    in_specs=[_row_spec(DIN), _full_spec((DIN, H)), _full_spec((1, H)),
              _row_spec(8), _row_spec(8)],
    out_specs=[_row_spec(H), _row_spec(H)],
    out_shape=[jax.ShapeDtypeStruct((N, H), jnp.float32),
               jax.ShapeDtypeStruct((N, H), jnp.float32)],
    compiler_params=_params,
)

_round = pl.pallas_call(
    _round_body,
    grid=(_GRID,),
    in_specs=[_row_spec(H), _row_spec(H), _row_spec(H),
              _row_spec(8), _row_spec(8),
              _full_spec((H, H)), _full_spec((1, H))],
    out_specs=[_row_spec(H), _row_spec(H)],
    out_shape=[jax.ShapeDtypeStruct((N, H), jnp.float32),
               jax.ShapeDtypeStruct((N, H), jnp.float32)],
    compiler_params=_params,
)

_final = pl.pallas_call(
    _final_body,
    grid=(_GRID,),
    in_specs=[_row_spec(H), _row_spec(H), _row_spec(H),
              _row_spec(8), _row_spec(8),
              _full_spec((H, H)), _full_spec((1, H)),
              _full_spec((H, DOUT)), _full_spec((1, DOUT))],
    out_specs=_row_spec(DOUT),
    out_shape=jax.ShapeDtypeStruct((N, DOUT), jnp.float32),
    compiler_params=_params,
)


def kernel(x, edge_index, W0, b0, W1, b1, W2, b2, W3, b3, W4, b4, W5, b5,
           W6, b6, W7, b7, W8, b8):
    src = edge_index[0]
    dst = edge_index[1]
    pad = EP - E
    pad_src = jnp.arange(pad, dtype=jnp.int32) % N
    pad_dst = N + jnp.arange(pad, dtype=jnp.int32) % (NPAD - N)
    srcp = jnp.concatenate([src, pad_src]).reshape(NW, NCHUNK, CHUNK)
    dstp = jnp.concatenate([dst, pad_dst]).reshape(NW, NCHUNK, CHUNK)
    zeros_h = jnp.zeros((NPAD, H), jnp.float32)
    zeros_8 = jnp.zeros((NPAD, 8), jnp.float32)
    ones_8 = jnp.ones((CHUNK, 8), jnp.float32)

    degp = _deg_call(dstp, ones_8, zeros_8)
    p0 = degp[0]
    p1 = degp[1]

    h, g = _mm0(x, W0, b0.reshape(1, H), p0, p1)
    for Wi, bi in ((W1, b1), (W2, b2), (W3, b3), (W4, b4), (W5, b5),
                   (W6, b6)):
        acc = _prop_call(g, srcp, dstp, zeros_h)
        h, g = _round(acc[0], acc[1], h, p0, p1, Wi, bi.reshape(1, H))
    acc = _prop_call(g, srcp, dstp, zeros_h)
    return _final(acc[0], acc[1], h, p0, p1, W7, b7.reshape(1, H),
                  W8, b8.reshape(1, DOUT))


# speedup vs baseline: 38.1391x; 1.1369x over previous
import functools

import jax
import jax.numpy as jnp
from jax import lax
from jax.experimental import pallas as pl
from jax.experimental.pallas import tpu as pltpu
from jax.experimental.pallas import tpu_sc as plsc

N = 10000
E = 320000
DIN = 128
H = 32
DOUT = 128

NC = 2
NS = 16
NW = NC * NS
CHUNK = 128
NCHUNK = 80
EPW = NCHUNK * CHUNK
EP = NW * EPW
NPAD = 10240
RPT = NPAD // NS

_mesh = plsc.VectorSubcoreMesh(core_axis_name="c", subcore_axis_name="s")
_sc_params = pltpu.CompilerParams(use_tc_tiling_on_sc=False)


def _deg_body(dsts_hbm, ones_hbm, zeros_hbm, out_hbm, didx, ones_v, acc, sem):
    c = lax.axis_index("c")
    s = lax.axis_index("s")
    wid = s * NC + c
    pltpu.sync_copy(zeros_hbm.at[pl.ds(s * RPT, RPT)],
                    acc.at[pl.ds(s * RPT, RPT)])
    pltpu.sync_copy(ones_hbm, ones_v)
    pltpu.sync_copy(dsts_hbm.at[wid], didx)
    plsc.subcore_barrier()

    def body(i, carry):
        for b in range(8):
            pltpu.async_copy(ones_v, acc.at[didx.at[8 * i + b]], sem,
                             add=True)
        for b in range(8):
            pltpu.make_async_copy(ones_v, acc.at[didx.at[0]], sem).wait()
        return carry

    lax.fori_loop(0, NCHUNK // 8, body, 0)
    plsc.subcore_barrier()
    pltpu.sync_copy(acc.at[pl.ds(s * RPT, RPT)],
                    out_hbm.at[c, pl.ds(s * RPT, RPT)])


_deg_call = pl.kernel(
    _deg_body,
    out_type=jax.ShapeDtypeStruct((NC, NPAD, 8), jnp.float32),
    mesh=_mesh,
    scratch_types=[
        pltpu.VMEM((NCHUNK, CHUNK), jnp.int32),
        pltpu.VMEM((CHUNK, 8), jnp.float32),
        pltpu.VMEM_SHARED((NPAD, 8), jnp.float32),
        pltpu.SemaphoreType.DMA,
    ],
    compiler_params=_sc_params,
)


def _prop_body(g_hbm, srcs_hbm, dsts_hbm, zeros_hbm, out_hbm,
               sidx, didx, rows, acc, gsa, gsb, ssa, ssb):
    c = lax.axis_index("c")
    s = lax.axis_index("s")
    wid = s * NC + c
    pltpu.sync_copy(zeros_hbm.at[pl.ds(s * RPT, RPT)],
                    acc.at[pl.ds(s * RPT, RPT)])
    pltpu.sync_copy(srcs_hbm.at[wid], sidx)
    pltpu.sync_copy(dsts_hbm.at[wid], didx)
    plsc.subcore_barrier()

    def _wait_g(sem, b):
        pltpu.make_async_copy(g_hbm.at[sidx.at[0]], rows.at[b], sem).wait()

    def _wait_s(sem, b):
        pltpu.make_async_copy(rows.at[b], acc.at[didx.at[0]], sem).wait()

    def body(i, carry):
        base = 8 * i

        @pl.when(i > 0)
        def _():
            for b in range(4):
                _wait_s(ssa, b)

        for b in range(4):
            pltpu.async_copy(g_hbm.at[sidx.at[base + b]], rows.at[b], gsa)
        for b in range(4):
            _wait_g(gsa, b)

        @pl.when(i > 0)
        def _():
            for b in range(4):
                _wait_s(ssb, 4 + b)

        for b in range(4):
            pltpu.async_copy(g_hbm.at[sidx.at[base + 4 + b]],
                             rows.at[4 + b], gsb)
        for b in range(4):
            pltpu.async_copy(rows.at[b], acc.at[didx.at[base + b]], ssa,
                             add=True)
        for b in range(4):
            _wait_g(gsb, 4 + b)
        for b in range(4):
            pltpu.async_copy(rows.at[4 + b], acc.at[didx.at[base + 4 + b]],
                             ssb, add=True)
        return carry

    lax.fori_loop(0, NCHUNK // 8, body, 0)
    for b in range(4):
        _wait_s(ssa, b)
    for b in range(4):
        _wait_s(ssb, 4 + b)
    plsc.subcore_barrier()
    pltpu.sync_copy(acc.at[pl.ds(s * RPT, RPT)],
                    out_hbm.at[c, pl.ds(s * RPT, RPT)])


_prop_call = pl.kernel(
    _prop_body,
    out_type=jax.ShapeDtypeStruct((NC, NPAD, H), jnp.float32),
    mesh=_mesh,
    scratch_types=[
        pltpu.VMEM((NCHUNK, CHUNK), jnp.int32),
        pltpu.VMEM((NCHUNK, CHUNK), jnp.int32),
        pltpu.VMEM((8, CHUNK, H), jnp.float32),
        pltpu.VMEM_SHARED((NPAD, H), jnp.float32),
        pltpu.SemaphoreType.DMA,
        pltpu.SemaphoreType.DMA,
        pltpu.SemaphoreType.DMA,
        pltpu.SemaphoreType.DMA,
    ],
    compiler_params=_sc_params,
)

_BM = 2000
_GRID = N // _BM


def _mm0_body(x_ref, w_ref, b_ref, dp0_ref, dp1_ref, h_ref, g_ref):
    h = jnp.dot(x_ref[...], w_ref[...],
                preferred_element_type=jnp.float32) + b_ref[...]
    deg = 1.0 + dp0_ref[0, :, :1] + dp1_ref[0, :, :1]
    dis = lax.rsqrt(deg)
    h_ref[...] = h
    g_ref[...] = h * dis


def _round_body(a0_ref, a1_ref, h_ref, dp0_ref, dp1_ref, w_ref, b_ref,
                ho_ref, go_ref):
    deg = 1.0 + dp0_ref[0, :, :1] + dp1_ref[0, :, :1]
    dis = lax.rsqrt(deg)
    prop = dis * (a0_ref[0] + a1_ref[0]) + h_ref[...] / deg
    h = jnp.maximum(
        jnp.dot(prop, w_ref[...], preferred_element_type=jnp.float32)
        + b_ref[...], 0.0)
    ho_ref[...] = h
    go_ref[...] = h * dis


def _final_body(a0_ref, a1_ref, h_ref, dp0_ref, dp1_ref, w7_ref, b7_ref,
                w8_ref, b8_ref, o_ref):
    deg = 1.0 + dp0_ref[0, :, :1] + dp1_ref[0, :, :1]
    dis = lax.rsqrt(deg)
    prop = dis * (a0_ref[0] + a1_ref[0]) + h_ref[...] / deg
    h7 = jnp.maximum(
        jnp.dot(prop, w7_ref[...], preferred_element_type=jnp.float32)
        + b7_ref[...], 0.0)
    o_ref[...] = jnp.dot(h7, w8_ref[...],
                         preferred_element_type=jnp.float32) + b8_ref[...]


def _row_spec(width):
    return pl.BlockSpec((_BM, width), lambda i: (i, 0))


def _slab_spec(width, c):
    return pl.BlockSpec((1, _BM, width), lambda i: (c, i, 0))


def _full_spec(shape):
    return pl.BlockSpec(shape, lambda i: (0,) * len(shape))


_params = pltpu.CompilerParams(dimension_semantics=("arbitrary",))

_mm0 = pl.pallas_call(
    _mm0_body,
    grid=(_GRID,),
    in_specs=[_row_spec(DIN), _full_spec((DIN, H)), _full_spec((1, H)),
              _slab_spec(8, 0), _slab_spec(8, 1)],
    out_specs=[_row_spec(H), _row_spec(H)],
    out_shape=[jax.ShapeDtypeStruct((N, H), jnp.float32),
               jax.ShapeDtypeStruct((N, H), jnp.float32)],
    compiler_params=_params,
)

_round = pl.pallas_call(
    _round_body,
    grid=(_GRID,),
    in_specs=[_slab_spec(H, 0), _slab_spec(H, 1), _row_spec(H),
              _slab_spec(8, 0), _slab_spec(8, 1),
              _full_spec((H, H)), _full_spec((1, H))],
    out_specs=[_row_spec(H), _row_spec(H)],
    out_shape=[jax.ShapeDtypeStruct((N, H), jnp.float32),
               jax.ShapeDtypeStruct((N, H), jnp.float32)],
    compiler_params=_params,
)

_final = pl.pallas_call(
    _final_body,
    grid=(_GRID,),
    in_specs=[_slab_spec(H, 0), _slab_spec(H, 1), _row_spec(H),
              _slab_spec(8, 0), _slab_spec(8, 1),
              _full_spec((H, H)), _full_spec((1, H)),
              _full_spec((H, DOUT)), _full_spec((1, DOUT))],
    out_specs=_row_spec(DOUT),
    out_shape=jax.ShapeDtypeStruct((N, DOUT), jnp.float32),
    compiler_params=_params,
)


def kernel(x, edge_index, W0, b0, W1, b1, W2, b2, W3, b3, W4, b4, W5, b5,
           W6, b6, W7, b7, W8, b8):
    src = edge_index[0]
    dst = edge_index[1]
    pad = EP - E
    pad_src = jnp.arange(pad, dtype=jnp.int32) % N
    pad_dst = N + jnp.arange(pad, dtype=jnp.int32) % (NPAD - N)
    srcp = jnp.concatenate([src, pad_src]).reshape(NW, NCHUNK, CHUNK)
    dstp = jnp.concatenate([dst, pad_dst]).reshape(NW, NCHUNK, CHUNK)
    zeros_h = jnp.zeros((NPAD, H), jnp.float32)
    zeros_8 = jnp.zeros((NPAD, 8), jnp.float32)
    ones_8 = jnp.ones((CHUNK, 8), jnp.float32)

    degp = _deg_call(dstp, ones_8, zeros_8)

    h, g = _mm0(x, W0, b0.reshape(1, H), degp, degp)
    for Wi, bi in ((W1, b1), (W2, b2), (W3, b3), (W4, b4), (W5, b5),
                   (W6, b6)):
        acc = _prop_call(g, srcp, dstp, zeros_h)
        h, g = _round(acc, acc, h, degp, degp, Wi, bi.reshape(1, H))
    acc = _prop_call(g, srcp, dstp, zeros_h)
    return _final(acc, acc, h, degp, degp, W7, b7.reshape(1, H),
                  W8, b8.reshape(1, DOUT))


# speedup vs baseline: 42.1007x; 1.1039x over previous
import functools

import jax
import jax.numpy as jnp
from jax import lax
from jax.experimental import pallas as pl
from jax.experimental.pallas import tpu as pltpu
from jax.experimental.pallas import tpu_sc as plsc

N = 10000
E = 320000
DIN = 128
H = 32
DOUT = 128

NC = 2
NS = 16
NW = NC * NS
CHUNK = 128
NCHUNK = 80
EPW = NCHUNK * CHUNK
EP = NW * EPW
NPAD = 10240
RPT = NPAD // NS

_mesh = plsc.VectorSubcoreMesh(core_axis_name="c", subcore_axis_name="s")
_sc_params = pltpu.CompilerParams(use_tc_tiling_on_sc=False)


def _deg_body(dsts_hbm, ones_hbm, zeros_hbm, out_hbm, didx, ones_v, acc, sem):
    c = lax.axis_index("c")
    s = lax.axis_index("s")
    wid = s * NC + c
    pltpu.sync_copy(zeros_hbm.at[pl.ds(s * RPT, RPT)],
                    acc.at[pl.ds(s * RPT, RPT)])
    pltpu.sync_copy(ones_hbm, ones_v)
    pltpu.sync_copy(dsts_hbm.at[wid], didx)
    plsc.subcore_barrier()

    def body(i, carry):
        for b in range(8):
            pltpu.async_copy(ones_v, acc.at[didx.at[8 * i + b]], sem,
                             add=True)
        for b in range(8):
            pltpu.make_async_copy(ones_v, acc.at[didx.at[0]], sem).wait()
        return carry

    lax.fori_loop(0, NCHUNK // 8, body, 0)
    plsc.subcore_barrier()
    pltpu.sync_copy(acc.at[pl.ds(s * RPT, RPT)],
                    out_hbm.at[c, pl.ds(s * RPT, RPT), pl.ds(0, 8)])


_deg_call = pl.kernel(
    _deg_body,
    out_type=jax.ShapeDtypeStruct((NC, NPAD, 128), jnp.float32),
    mesh=_mesh,
    scratch_types=[
        pltpu.VMEM((NCHUNK, CHUNK), jnp.int32),
        pltpu.VMEM((CHUNK, 8), jnp.float32),
        pltpu.VMEM_SHARED((NPAD, 8), jnp.float32),
        pltpu.SemaphoreType.DMA,
    ],
    compiler_params=_sc_params,
)


def _prop_body(g_hbm, srcs_hbm, dsts_hbm, zeros_hbm, out_hbm,
               sidx, didx, rows, acc, gsa, gsb, ssa, ssb):
    c = lax.axis_index("c")
    s = lax.axis_index("s")
    wid = s * NC + c
    pltpu.sync_copy(zeros_hbm.at[pl.ds(s * RPT, RPT)],
                    acc.at[pl.ds(s * RPT, RPT)])
    pltpu.sync_copy(srcs_hbm.at[wid], sidx)
    pltpu.sync_copy(dsts_hbm.at[wid], didx)
    plsc.subcore_barrier()

    def _wait_g(sem, b):
        pltpu.make_async_copy(g_hbm.at[sidx.at[0]], rows.at[b], sem).wait()

    def _wait_s(sem, b):
        pltpu.make_async_copy(rows.at[b], acc.at[didx.at[0]], sem).wait()

    def body(i, carry):
        base = 8 * i

        @pl.when(i > 0)
        def _():
            for b in range(4):
                _wait_s(ssa, b)

        for b in range(4):
            pltpu.async_copy(g_hbm.at[sidx.at[base + b]], rows.at[b], gsa)
        for b in range(4):
            _wait_g(gsa, b)

        @pl.when(i > 0)
        def _():
            for b in range(4):
                _wait_s(ssb, 4 + b)

        for b in range(4):
            pltpu.async_copy(g_hbm.at[sidx.at[base + 4 + b]],
                             rows.at[4 + b], gsb)
        for b in range(4):
            pltpu.async_copy(rows.at[b], acc.at[didx.at[base + b]], ssa,
                             add=True)
        for b in range(4):
            _wait_g(gsb, 4 + b)
        for b in range(4):
            pltpu.async_copy(rows.at[4 + b], acc.at[didx.at[base + 4 + b]],
                             ssb, add=True)
        return carry

    lax.fori_loop(0, NCHUNK // 8, body, 0)
    for b in range(4):
        _wait_s(ssa, b)
    for b in range(4):
        _wait_s(ssb, 4 + b)
    plsc.subcore_barrier()
    pltpu.sync_copy(acc.at[pl.ds(s * RPT, RPT)],
                    out_hbm.at[c, pl.ds(s * RPT, RPT), pl.ds(0, H)])


_prop_call = pl.kernel(
    _prop_body,
    out_type=jax.ShapeDtypeStruct((NC, NPAD, 128), jnp.float32),
    mesh=_mesh,
    scratch_types=[
        pltpu.VMEM((NCHUNK, CHUNK), jnp.int32),
        pltpu.VMEM((NCHUNK, CHUNK), jnp.int32),
        pltpu.VMEM((8, CHUNK, H), jnp.float32),
        pltpu.VMEM_SHARED((NPAD, H), jnp.float32),
        pltpu.SemaphoreType.DMA,
        pltpu.SemaphoreType.DMA,
        pltpu.SemaphoreType.DMA,
        pltpu.SemaphoreType.DMA,
    ],
    compiler_params=_sc_params,
)

_BM = 2000
_GRID = N // _BM


def _mm0_body(x_ref, w_ref, b_ref, dp0_ref, dp1_ref, h_ref, g_ref):
    h = jnp.dot(x_ref[...], w_ref[...],
                preferred_element_type=jnp.float32) + b_ref[...]
    deg = 1.0 + dp0_ref[0, :, :1] + dp1_ref[0, :, :1]
    dis = lax.rsqrt(deg)
    h_ref[...] = h
    g_ref[...] = h * dis


def _round_body(a0_ref, a1_ref, h_ref, dp0_ref, dp1_ref, w_ref, b_ref,
                ho_ref, go_ref):
    deg = 1.0 + dp0_ref[0, :, :1] + dp1_ref[0, :, :1]
    dis = lax.rsqrt(deg)
    prop = dis * (a0_ref[0, :, :H] + a1_ref[0, :, :H]) + h_ref[...] / deg
    h = jnp.maximum(
        jnp.dot(prop, w_ref[...], preferred_element_type=jnp.float32)
        + b_ref[...], 0.0)
    ho_ref[...] = h
    go_ref[...] = h * dis


def _final_body(a0_ref, a1_ref, h_ref, dp0_ref, dp1_ref, w7_ref, b7_ref,
                w8_ref, b8_ref, o_ref):
    deg = 1.0 + dp0_ref[0, :, :1] + dp1_ref[0, :, :1]
    dis = lax.rsqrt(deg)
    prop = dis * (a0_ref[0, :, :H] + a1_ref[0, :, :H]) + h_ref[...] / deg
    h7 = jnp.maximum(
        jnp.dot(prop, w7_ref[...], preferred_element_type=jnp.float32)
        + b7_ref[...], 0.0)
    o_ref[...] = jnp.dot(h7, w8_ref[...],
                         preferred_element_type=jnp.float32) + b8_ref[...]


def _row_spec(width):
    return pl.BlockSpec((_BM, width), lambda i: (i, 0))


def _slab_spec(width, c):
    return pl.BlockSpec((1, _BM, width), lambda i: (c, i, 0))


def _full_spec(shape):
    return pl.BlockSpec(shape, lambda i: (0,) * len(shape))


_params = pltpu.CompilerParams(dimension_semantics=("arbitrary",))

_g_spec = _row_spec(H)

_mm0 = pl.pallas_call(
    _mm0_body,
    grid=(_GRID,),
    in_specs=[_row_spec(DIN), _full_spec((DIN, H)), _full_spec((1, H)),
              _slab_spec(128, 0), _slab_spec(128, 1)],
    out_specs=[_row_spec(H), _g_spec],
    out_shape=[jax.ShapeDtypeStruct((N, H), jnp.float32),
               jax.ShapeDtypeStruct((N, H), jnp.float32)],
    compiler_params=_params,
)

_round = pl.pallas_call(
    _round_body,
    grid=(_GRID,),
    in_specs=[_slab_spec(128, 0), _slab_spec(128, 1), _row_spec(H),
              _slab_spec(128, 0), _slab_spec(128, 1),
              _full_spec((H, H)), _full_spec((1, H))],
    out_specs=[_row_spec(H), _g_spec],
    out_shape=[jax.ShapeDtypeStruct((N, H), jnp.float32),
               jax.ShapeDtypeStruct((N, H), jnp.float32)],
    compiler_params=_params,
)

_final = pl.pallas_call(
    _final_body,
    grid=(_GRID,),
    in_specs=[_slab_spec(128, 0), _slab_spec(128, 1), _row_spec(H),
              _slab_spec(128, 0), _slab_spec(128, 1),
              _full_spec((H, H)), _full_spec((1, H)),
              _full_spec((H, DOUT)), _full_spec((1, DOUT))],
    out_specs=_row_spec(DOUT),
    out_shape=jax.ShapeDtypeStruct((N, DOUT), jnp.float32),
    compiler_params=_params,
)


def kernel(x, edge_index, W0, b0, W1, b1, W2, b2, W3, b3, W4, b4, W5, b5,
           W6, b6, W7, b7, W8, b8):
    src = edge_index[0]
    dst = edge_index[1]
    pad = EP - E
    pad_src = jnp.arange(pad, dtype=jnp.int32) % N
    pad_dst = N + jnp.arange(pad, dtype=jnp.int32) % (NPAD - N)
    srcp = jnp.concatenate([src, pad_src]).reshape(NW, NCHUNK, CHUNK)
    dstp = jnp.concatenate([dst, pad_dst]).reshape(NW, NCHUNK, CHUNK)
    zeros_h = jnp.zeros((NPAD, H), jnp.float32)
    zeros_8 = jnp.zeros((NPAD, 8), jnp.float32)
    ones_8 = jnp.ones((CHUNK, 8), jnp.float32)

    degp = _deg_call(dstp, ones_8, zeros_8)

    h, g = _mm0(x, W0, b0.reshape(1, H), degp, degp)
    for Wi, bi in ((W1, b1), (W2, b2), (W3, b3), (W4, b4), (W5, b5),
                   (W6, b6)):
        acc = _prop_call(g, srcp, dstp, zeros_h)
        h, g = _round(acc, acc, h, degp, degp, Wi, bi.reshape(1, H))
    acc = _prop_call(g, srcp, dstp, zeros_h)
    return _final(acc, acc, h, degp, degp, W7, b7.reshape(1, H),
                  W8, b8.reshape(1, DOUT))


# speedup vs baseline: 44.4571x; 1.0560x over previous
import functools

import jax
import jax.numpy as jnp
from jax import lax
from jax.experimental import pallas as pl
from jax.experimental.pallas import tpu as pltpu
from jax.experimental.pallas import tpu_sc as plsc

N = 10000
E = 320000
DIN = 128
H = 32
DOUT = 128

NC = 2
NS = 16
NW = NC * NS
CHUNK = 128
NCHUNK = 80
EPW = NCHUNK * CHUNK
EP = NW * EPW
NPAD = 10240
RPT = NPAD // NS

_mesh = plsc.VectorSubcoreMesh(core_axis_name="c", subcore_axis_name="s")
_sc_params = pltpu.CompilerParams(use_tc_tiling_on_sc=False)


def _deg_body(dsts_hbm, ones_hbm, zeros_hbm, out_hbm, didx, ones_v, acc, sem):
    c = lax.axis_index("c")
    s = lax.axis_index("s")
    wid = s * NC + c
    pltpu.sync_copy(zeros_hbm.at[pl.ds(s * RPT, RPT)],
                    acc.at[pl.ds(s * RPT, RPT)])
    pltpu.sync_copy(ones_hbm, ones_v)
    pltpu.sync_copy(dsts_hbm.at[wid], didx)
    plsc.subcore_barrier()

    def body(i, carry):
        for b in range(8):
            pltpu.async_copy(ones_v, acc.at[didx.at[8 * i + b]], sem,
                             add=True)
        for b in range(8):
            pltpu.make_async_copy(ones_v, acc.at[didx.at[0]], sem).wait()
        return carry

    lax.fori_loop(0, NCHUNK // 8, body, 0)
    plsc.subcore_barrier()
    pltpu.sync_copy(acc.at[pl.ds(s * RPT, RPT)],
                    out_hbm.at[c, pl.ds(s * RPT, RPT), pl.ds(0, 8)])


_deg_call = pl.kernel(
    _deg_body,
    out_type=jax.ShapeDtypeStruct((NC, NPAD, 128), jnp.float32),
    mesh=_mesh,
    scratch_types=[
        pltpu.VMEM((NCHUNK, CHUNK), jnp.int32),
        pltpu.VMEM((CHUNK, 8), jnp.float32),
        pltpu.VMEM_SHARED((NPAD, 8), jnp.float32),
        pltpu.SemaphoreType.DMA,
    ],
    compiler_params=_sc_params,
)


def _prop_body(g_hbm, srcs_hbm, dsts_hbm, zeros_hbm, out_hbm,
               sidx, didx, rows, acc, gsa, gsb, ssa, ssb):
    c = lax.axis_index("c")
    s = lax.axis_index("s")
    wid = s * NC + c
    pltpu.sync_copy(zeros_hbm.at[pl.ds(s * RPT, RPT)],
                    acc.at[pl.ds(s * RPT, RPT)])
    pltpu.sync_copy(srcs_hbm.at[wid], sidx)
    pltpu.sync_copy(dsts_hbm.at[wid], didx)
    plsc.subcore_barrier()

    def _wait_g(sem, b):
        pltpu.make_async_copy(g_hbm.at[sidx.at[0]], rows.at[b], sem).wait()

    def _wait_s(sem, b):
        pltpu.make_async_copy(rows.at[b], acc.at[didx.at[0]], sem).wait()

    def body(i, carry):
        base = 16 * i

        @pl.when(i > 0)
        def _():
            for b in range(8):
                _wait_s(ssa, b)

        for b in range(8):
            pltpu.async_copy(g_hbm.at[sidx.at[base + b]], rows.at[b], gsa)
        for b in range(8):
            _wait_g(gsa, b)

        @pl.when(i > 0)
        def _():
            for b in range(8):
                _wait_s(ssb, 8 + b)

        for b in range(8):
            pltpu.async_copy(g_hbm.at[sidx.at[base + 8 + b]],
                             rows.at[8 + b], gsb)
        for b in range(8):
            pltpu.async_copy(rows.at[b], acc.at[didx.at[base + b]], ssa,
                             add=True)
        for b in range(8):
            _wait_g(gsb, 8 + b)
        for b in range(8):
            pltpu.async_copy(rows.at[8 + b], acc.at[didx.at[base + 8 + b]],
                             ssb, add=True)
        return carry

    lax.fori_loop(0, NCHUNK // 16, body, 0)
    for b in range(8):
        _wait_s(ssa, b)
    for b in range(8):
        _wait_s(ssb, 8 + b)
    plsc.subcore_barrier()
    pltpu.sync_copy(acc.at[pl.ds(s * RPT, RPT)],
                    out_hbm.at[c, pl.ds(s * RPT, RPT), pl.ds(0, H)])


_prop_call = pl.kernel(
    _prop_body,
    out_type=jax.ShapeDtypeStruct((NC, NPAD, 128), jnp.float32),
    mesh=_mesh,
    scratch_types=[
        pltpu.VMEM((NCHUNK, CHUNK), jnp.int32),
        pltpu.VMEM((NCHUNK, CHUNK), jnp.int32),
        pltpu.VMEM((16, CHUNK, H), jnp.float32),
        pltpu.VMEM_SHARED((NPAD, H), jnp.float32),
        pltpu.SemaphoreType.DMA,
        pltpu.SemaphoreType.DMA,
        pltpu.SemaphoreType.DMA,
        pltpu.SemaphoreType.DMA,
    ],
    compiler_params=_sc_params,
)

_BM = 2000
_GRID = N // _BM


def _mm0_body(x_ref, w_ref, b_ref, dp0_ref, dp1_ref, h_ref, g_ref):
    h = jnp.dot(x_ref[...], w_ref[...],
                preferred_element_type=jnp.float32) + b_ref[...]
    deg = 1.0 + dp0_ref[0, :, :1] + dp1_ref[0, :, :1]
    dis = lax.rsqrt(deg)
    h_ref[...] = h
    g_ref[...] = h * dis


def _round_body(a0_ref, a1_ref, h_ref, dp0_ref, dp1_ref, w_ref, b_ref,
                ho_ref, go_ref):
    deg = 1.0 + dp0_ref[0, :, :1] + dp1_ref[0, :, :1]
    dis = lax.rsqrt(deg)
    prop = dis * (a0_ref[0, :, :H] + a1_ref[0, :, :H]) + h_ref[...] / deg
    h = jnp.maximum(
        jnp.dot(prop, w_ref[...], preferred_element_type=jnp.float32)
        + b_ref[...], 0.0)
    ho_ref[...] = h
    go_ref[...] = h * dis


def _final_body(a0_ref, a1_ref, h_ref, dp0_ref, dp1_ref, w7_ref, b7_ref,
                w8_ref, b8_ref, o_ref):
    deg = 1.0 + dp0_ref[0, :, :1] + dp1_ref[0, :, :1]
    dis = lax.rsqrt(deg)
    prop = dis * (a0_ref[0, :, :H] + a1_ref[0, :, :H]) + h_ref[...] / deg
    h7 = jnp.maximum(
        jnp.dot(prop, w7_ref[...], preferred_element_type=jnp.float32)
        + b7_ref[...], 0.0)
    o_ref[...] = jnp.dot(h7, w8_ref[...],
                         preferred_element_type=jnp.float32) + b8_ref[...]


def _row_spec(width):
    return pl.BlockSpec((_BM, width), lambda i: (i, 0))


def _slab_spec(width, c):
    return pl.BlockSpec((1, _BM, width), lambda i: (c, i, 0))


def _full_spec(shape):
    return pl.BlockSpec(shape, lambda i: (0,) * len(shape))


_params = pltpu.CompilerParams(dimension_semantics=("arbitrary",))

_g_spec = _row_spec(H)

_mm0 = pl.pallas_call(
    _mm0_body,
    grid=(_GRID,),
    in_specs=[_row_spec(DIN), _full_spec((DIN, H)), _full_spec((1, H)),
              _slab_spec(128, 0), _slab_spec(128, 1)],
    out_specs=[_row_spec(H), _g_spec],
    out_shape=[jax.ShapeDtypeStruct((N, H), jnp.float32),
               jax.ShapeDtypeStruct((N, H), jnp.float32)],
    compiler_params=_params,
)

_round = pl.pallas_call(
    _round_body,
    grid=(_GRID,),
    in_specs=[_slab_spec(128, 0), _slab_spec(128, 1), _row_spec(H),
              _slab_spec(128, 0), _slab_spec(128, 1),
              _full_spec((H, H)), _full_spec((1, H))],
    out_specs=[_row_spec(H), _g_spec],
    out_shape=[jax.ShapeDtypeStruct((N, H), jnp.float32),
               jax.ShapeDtypeStruct((N, H), jnp.float32)],
    compiler_params=_params,
)

_final = pl.pallas_call(
    _final_body,
    grid=(_GRID,),
    in_specs=[_slab_spec(128, 0), _slab_spec(128, 1), _row_spec(H),
              _slab_spec(128, 0), _slab_spec(128, 1),
              _full_spec((H, H)), _full_spec((1, H)),
              _full_spec((H, DOUT)), _full_spec((1, DOUT))],
    out_specs=_row_spec(DOUT),
    out_shape=jax.ShapeDtypeStruct((N, DOUT), jnp.float32),
    compiler_params=_params,
)


def kernel(x, edge_index, W0, b0, W1, b1, W2, b2, W3, b3, W4, b4, W5, b5,
           W6, b6, W7, b7, W8, b8):
    src = edge_index[0]
    dst = edge_index[1]
    pad = EP - E
    pad_src = jnp.arange(pad, dtype=jnp.int32) % N
    pad_dst = N + jnp.arange(pad, dtype=jnp.int32) % (NPAD - N)
    srcp = jnp.concatenate([src, pad_src]).reshape(NW, NCHUNK, CHUNK)
    dstp = jnp.concatenate([dst, pad_dst]).reshape(NW, NCHUNK, CHUNK)
    zeros_h = jnp.zeros((NPAD, H), jnp.float32)
    zeros_8 = jnp.zeros((NPAD, 8), jnp.float32)
    ones_8 = jnp.ones((CHUNK, 8), jnp.float32)

    degp = _deg_call(dstp, ones_8, zeros_8)

    h, g = _mm0(x, W0, b0.reshape(1, H), degp, degp)
    for Wi, bi in ((W1, b1), (W2, b2), (W3, b3), (W4, b4), (W5, b5),
                   (W6, b6)):
        acc = _prop_call(g, srcp, dstp, zeros_h)
        h, g = _round(acc, acc, h, degp, degp, Wi, bi.reshape(1, H))
    acc = _prop_call(g, srcp, dstp, zeros_h)
    return _final(acc, acc, h, degp, degp, W7, b7.reshape(1, H),
                  W8, b8.reshape(1, DOUT))


# speedup vs baseline: 46.6285x; 1.0488x over previous
import functools

import jax
import jax.numpy as jnp
from jax import lax
from jax.experimental import pallas as pl
from jax.experimental.pallas import tpu as pltpu
from jax.experimental.pallas import tpu_sc as plsc

N = 10000
E = 320000
DIN = 128
H = 32
DOUT = 128

NC = 2
NS = 16
NW = NC * NS
CHUNK = 128
NCHUNK = 80
EPW = NCHUNK * CHUNK
EP = NW * EPW
NPAD = 10240
RPT = NPAD // NS

_mesh = plsc.VectorSubcoreMesh(core_axis_name="c", subcore_axis_name="s")
_sc_params = pltpu.CompilerParams(use_tc_tiling_on_sc=False)


def _deg_body(dsts_hbm, ones_hbm, zeros_hbm, out_hbm, didx, ones_v, acc, sem):
    c = lax.axis_index("c")
    s = lax.axis_index("s")
    wid = s * NC + c
    pltpu.sync_copy(zeros_hbm.at[pl.ds(s * RPT, RPT)],
                    acc.at[pl.ds(s * RPT, RPT)])
    pltpu.sync_copy(ones_hbm, ones_v)
    pltpu.sync_copy(dsts_hbm.at[wid], didx)
    plsc.subcore_barrier()

    def body(i, carry):
        for b in range(8):
            pltpu.async_copy(ones_v, acc.at[didx.at[8 * i + b]], sem,
                             add=True)
        for b in range(8):
            pltpu.make_async_copy(ones_v, acc.at[didx.at[0]], sem).wait()
        return carry

    lax.fori_loop(0, NCHUNK // 8, body, 0)
    plsc.subcore_barrier()
    pltpu.sync_copy(acc.at[pl.ds(s * RPT, RPT)],
                    out_hbm.at[c, pl.ds(s * RPT, RPT), pl.ds(0, 8)])


_deg_call = pl.kernel(
    _deg_body,
    out_type=jax.ShapeDtypeStruct((NC, NPAD, 128), jnp.float32),
    mesh=_mesh,
    scratch_types=[
        pltpu.VMEM((NCHUNK, CHUNK), jnp.int32),
        pltpu.VMEM((CHUNK, 8), jnp.float32),
        pltpu.VMEM_SHARED((NPAD, 8), jnp.float32),
        pltpu.SemaphoreType.DMA,
    ],
    compiler_params=_sc_params,
)


def _prop_body(g_hbm, srcs_hbm, dsts_hbm, zeros_hbm, out_hbm,
               sidx, didx, rows, acc, gsa, gsb, ssa, ssb):
    c = lax.axis_index("c")
    s = lax.axis_index("s")
    wid = s * NC + c

    @pl.when(c == 0)
    def _():
        pltpu.sync_copy(g_hbm.at[pl.ds(s * RPT, RPT)],
                        acc.at[pl.ds(s * RPT, RPT)])

    @pl.when(c != 0)
    def _():
        pltpu.sync_copy(zeros_hbm.at[pl.ds(s * RPT, RPT)],
                        acc.at[pl.ds(s * RPT, RPT)])

    pltpu.sync_copy(srcs_hbm.at[wid], sidx)
    pltpu.sync_copy(dsts_hbm.at[wid], didx)
    plsc.subcore_barrier()

    def _wait_g(sem, b):
        pltpu.make_async_copy(g_hbm.at[sidx.at[0]], rows.at[b], sem).wait()

    def _wait_s(sem, b):
        pltpu.make_async_copy(rows.at[b], acc.at[didx.at[0]], sem).wait()

    def body(i, carry):
        base = 16 * i

        @pl.when(i > 0)
        def _():
            for b in range(8):
                _wait_s(ssa, b)

        for b in range(8):
            pltpu.async_copy(g_hbm.at[sidx.at[base + b]], rows.at[b], gsa)
        for b in range(8):
            _wait_g(gsa, b)

        @pl.when(i > 0)
        def _():
            for b in range(8):
                _wait_s(ssb, 8 + b)

        for b in range(8):
            pltpu.async_copy(g_hbm.at[sidx.at[base + 8 + b]],
                             rows.at[8 + b], gsb)
        for b in range(8):
            pltpu.async_copy(rows.at[b], acc.at[didx.at[base + b]], ssa,
                             add=True)
        for b in range(8):
            _wait_g(gsb, 8 + b)
        for b in range(8):
            pltpu.async_copy(rows.at[8 + b], acc.at[didx.at[base + 8 + b]],
                             ssb, add=True)
        return carry

    lax.fori_loop(0, NCHUNK // 16, body, 0)
    for b in range(8):
        _wait_s(ssa, b)
    for b in range(8):
        _wait_s(ssb, 8 + b)
    plsc.subcore_barrier()
    pltpu.sync_copy(acc.at[pl.ds(s * RPT, RPT)],
                    out_hbm.at[c, pl.ds(s * RPT, RPT), pl.ds(0, H)])


_prop_call = pl.kernel(
    _prop_body,
    out_type=jax.ShapeDtypeStruct((NC, NPAD, 128), jnp.float32),
    mesh=_mesh,
    scratch_types=[
        pltpu.VMEM((NCHUNK, CHUNK), jnp.int32),
        pltpu.VMEM((NCHUNK, CHUNK), jnp.int32),
        pltpu.VMEM((16, CHUNK, H), jnp.float32),
        pltpu.VMEM_SHARED((NPAD, H), jnp.float32),
        pltpu.SemaphoreType.DMA,
        pltpu.SemaphoreType.DMA,
        pltpu.SemaphoreType.DMA,
        pltpu.SemaphoreType.DMA,
    ],
    compiler_params=_sc_params,
)

def _mm0_body(x_ref, w_ref, b_ref, dis_ref, z_ref):
    h = jnp.dot(x_ref[...], w_ref[...],
                preferred_element_type=jnp.float32) + b_ref[...]
    z_ref[...] = h * dis_ref[...]


def _dis_body(dp0_ref, dp1_ref, dis_ref):
    deg = 1.0 + dp0_ref[0, :, :1] + dp1_ref[0, :, :1]
    dis_ref[...] = jnp.broadcast_to(lax.rsqrt(deg), (_BMR, H))


def _round_body(a0_ref, a1_ref, dis_ref, w_ref, b_ref, zo_ref):
    dis = dis_ref[...]
    prop = dis * (a0_ref[0, :, :H] + a1_ref[0, :, :H])
    h = jnp.maximum(
        jnp.dot(prop, w_ref[...], preferred_element_type=jnp.float32)
        + b_ref[...], 0.0)
    zo_ref[...] = h * dis


def _final_body(a0_ref, a1_ref, dis_ref, w7_ref, b7_ref,
                w8_ref, b8_ref, o_ref):
    prop = dis_ref[...] * (a0_ref[0, :, :H] + a1_ref[0, :, :H])
    h7 = jnp.maximum(
        jnp.dot(prop, w7_ref[...], preferred_element_type=jnp.float32)
        + b7_ref[...], 0.0)
    o_ref[...] = jnp.dot(h7, w8_ref[...],
                         preferred_element_type=jnp.float32) + b8_ref[...]


_BMR = 2560
_GRIDR = NPAD // _BMR
_BM = 2000
_GRIDF = N // _BM


def _rspec(rows, width):
    return pl.BlockSpec((rows, width), lambda i: (i, 0))


def _sspec(rows, c):
    return pl.BlockSpec((1, rows, 128), lambda i: (c, i, 0))


def _full_spec(shape):
    return pl.BlockSpec(shape, lambda i: (0,) * len(shape))


_params = pltpu.CompilerParams(dimension_semantics=("arbitrary",))

_mm0 = pl.pallas_call(
    _mm0_body,
    grid=(_GRIDR,),
    in_specs=[_rspec(_BMR, DIN), _full_spec((DIN, H)), _full_spec((1, H)),
              _rspec(_BMR, H)],
    out_specs=_rspec(_BMR, H),
    out_shape=jax.ShapeDtypeStruct((NPAD, H), jnp.float32),
    compiler_params=_params,
)

_dis_call = pl.pallas_call(
    _dis_body,
    grid=(_GRIDR,),
    in_specs=[_sspec(_BMR, 0), _sspec(_BMR, 1)],
    out_specs=_rspec(_BMR, H),
    out_shape=jax.ShapeDtypeStruct((NPAD, H), jnp.float32),
    compiler_params=_params,
)

_round = pl.pallas_call(
    _round_body,
    grid=(_GRIDR,),
    in_specs=[_sspec(_BMR, 0), _sspec(_BMR, 1), _rspec(_BMR, H),
              _full_spec((H, H)), _full_spec((1, H))],
    out_specs=_rspec(_BMR, H),
    out_shape=jax.ShapeDtypeStruct((NPAD, H), jnp.float32),
    compiler_params=_params,
)

_final = pl.pallas_call(
    _final_body,
    grid=(_GRIDF,),
    in_specs=[_sspec(_BM, 0), _sspec(_BM, 1), _rspec(_BM, H),
              _full_spec((H, H)), _full_spec((1, H)),
              _full_spec((H, DOUT)), _full_spec((1, DOUT))],
    out_specs=_rspec(_BM, DOUT),
    out_shape=jax.ShapeDtypeStruct((N, DOUT), jnp.float32),
    compiler_params=_params,
)


def kernel(x, edge_index, W0, b0, W1, b1, W2, b2, W3, b3, W4, b4, W5, b5,
           W6, b6, W7, b7, W8, b8):
    src = edge_index[0]
    dst = edge_index[1]
    pad = EP - E
    pad_src = jnp.arange(pad, dtype=jnp.int32) % N
    pad_dst = N + jnp.arange(pad, dtype=jnp.int32) % (NPAD - N)
    srcp = jnp.concatenate([src, pad_src]).reshape(NW, NCHUNK, CHUNK)
    dstp = jnp.concatenate([dst, pad_dst]).reshape(NW, NCHUNK, CHUNK)
    zeros_h = jnp.zeros((NPAD, H), jnp.float32)
    zeros_8 = jnp.zeros((NPAD, 8), jnp.float32)
    ones_8 = jnp.ones((CHUNK, 8), jnp.float32)
    x_pad = jnp.concatenate(
        [x, jnp.zeros((NPAD - N, DIN), jnp.float32)], axis=0)

    degp = _deg_call(dstp, ones_8, zeros_8)
    dis = _dis_call(degp, degp)

    z = _mm0(x_pad, W0, b0.reshape(1, H), dis)
    for Wi, bi in ((W1, b1), (W2, b2), (W3, b3), (W4, b4), (W5, b5),
                   (W6, b6)):
        acc = _prop_call(z, srcp, dstp, zeros_h)
        z = _round(acc, acc, dis, Wi, bi.reshape(1, H))
    acc = _prop_call(z, srcp, dstp, zeros_h)
    return _final(acc, acc, dis, W7, b7.reshape(1, H),
                  W8, b8.reshape(1, DOUT))


# speedup vs baseline: 49.9324x; 1.0709x over previous
import functools

import jax
import jax.numpy as jnp
from jax import lax
from jax.experimental import pallas as pl
from jax.experimental.pallas import tpu as pltpu
from jax.experimental.pallas import tpu_sc as plsc

N = 10000
E = 320000
DIN = 128
H = 32
DOUT = 128

NC = 2
NS = 16
NW = NC * NS
CHUNK = 128
NCHUNK = 80
EPW = NCHUNK * CHUNK
EP = NW * EPW
NPAD = 10240
RPT = NPAD // NS

_mesh = plsc.VectorSubcoreMesh(core_axis_name="c", subcore_axis_name="s")
_sc_params = pltpu.CompilerParams(use_tc_tiling_on_sc=False)


def _deg_body(dsts_hbm, ones_hbm, zeros_hbm, out_hbm, didx, ones_v, acc, sem):
    c = lax.axis_index("c")
    s = lax.axis_index("s")
    wid = s * NC + c
    pltpu.sync_copy(zeros_hbm.at[pl.ds(s * RPT, RPT)],
                    acc.at[pl.ds(s * RPT, RPT)])
    pltpu.sync_copy(ones_hbm, ones_v)
    pltpu.sync_copy(dsts_hbm.at[wid], didx)
    plsc.subcore_barrier()

    def body(i, carry):
        for b in range(8):
            pltpu.async_copy(ones_v, acc.at[didx.at[8 * i + b]], sem,
                             add=True)
        for b in range(8):
            pltpu.make_async_copy(ones_v, acc.at[didx.at[0]], sem).wait()
        return carry

    lax.fori_loop(0, NCHUNK // 8, body, 0)
    plsc.subcore_barrier()
    pltpu.sync_copy(acc.at[pl.ds(s * RPT, RPT)],
                    out_hbm.at[c, pl.ds(s * RPT, RPT), pl.ds(0, 8)])


_deg_call = pl.kernel(
    _deg_body,
    out_type=jax.ShapeDtypeStruct((NC, NPAD, 128), jnp.float32),
    mesh=_mesh,
    scratch_types=[
        pltpu.VMEM((NCHUNK, CHUNK), jnp.int32),
        pltpu.VMEM((CHUNK, 8), jnp.float32),
        pltpu.VMEM_SHARED((NPAD, 8), jnp.float32),
        pltpu.SemaphoreType.DMA,
    ],
    compiler_params=_sc_params,
)


def _prop_body(g_hbm, srcs_hbm, dsts_hbm, zeros_hbm, out_hbm,
               sidx, didx, rows, acc, gsa, gsb, ssa, ssb):
    c = lax.axis_index("c")
    s = lax.axis_index("s")
    wid = s * NC + c
    pltpu.sync_copy(zeros_hbm.at[pl.ds(s * RPT, RPT)],
                    acc.at[pl.ds(s * RPT, RPT)])
    pltpu.sync_copy(srcs_hbm.at[wid], sidx)
    pltpu.sync_copy(dsts_hbm.at[wid], didx)
    plsc.subcore_barrier()

    def _wait_g(sem, b):
        pltpu.make_async_copy(g_hbm.at[sidx.at[0]], rows.at[b], sem).wait()

    def _wait_s(sem, b):
        pltpu.make_async_copy(rows.at[b], acc.at[didx.at[0]], sem).wait()

    def body(i, carry):
        base = 16 * i

        @pl.when(i > 0)
        def _():
            for b in range(8):
                _wait_s(ssa, b)

        for b in range(8):
            pltpu.async_copy(g_hbm.at[sidx.at[base + b]], rows.at[b], gsa)
        for b in range(8):
            _wait_g(gsa, b)

        @pl.when(i > 0)
        def _():
            for b in range(8):
                _wait_s(ssb, 8 + b)

        for b in range(8):
            pltpu.async_copy(g_hbm.at[sidx.at[base + 8 + b]],
                             rows.at[8 + b], gsb)
        for b in range(8):
            pltpu.async_copy(rows.at[b], acc.at[didx.at[base + b]], ssa,
                             add=True)
        for b in range(8):
            _wait_g(gsb, 8 + b)
        for b in range(8):
            pltpu.async_copy(rows.at[8 + b], acc.at[didx.at[base + 8 + b]],
                             ssb, add=True)
        return carry

    lax.fori_loop(0, NCHUNK // 16, body, 0)
    for b in range(8):
        _wait_s(ssa, b)
    for b in range(8):
        _wait_s(ssb, 8 + b)
    plsc.subcore_barrier()
    pltpu.sync_copy(acc.at[pl.ds(s * RPT, RPT)],
                    out_hbm.at[c, pl.ds(s * RPT, RPT), pl.ds(0, H)])


_prop_call = pl.kernel(
    _prop_body,
    out_type=jax.ShapeDtypeStruct((NC, NPAD, 128), jnp.float32),
    mesh=_mesh,
    scratch_types=[
        pltpu.VMEM((NCHUNK, CHUNK), jnp.int32),
        pltpu.VMEM((NCHUNK, CHUNK), jnp.int32),
        pltpu.VMEM((16, CHUNK, H), jnp.float32),
        pltpu.VMEM_SHARED((NPAD, H), jnp.float32),
        pltpu.SemaphoreType.DMA,
        pltpu.SemaphoreType.DMA,
        pltpu.SemaphoreType.DMA,
        pltpu.SemaphoreType.DMA,
    ],
    compiler_params=_sc_params,
)

def _mm0_body(x_ref, w_ref, b_ref, dis_ref, z_ref):
    h = jnp.dot(x_ref[...], w_ref[...],
                preferred_element_type=jnp.float32) + b_ref[...]
    z = h * dis_ref[...]
    z_ref[...] = jnp.concatenate([z, z, z, z], axis=1)


def _dis_body(dp0_ref, dp1_ref, dis_ref):
    deg = 1.0 + dp0_ref[0, :, :1] + dp1_ref[0, :, :1]
    dis_ref[...] = jnp.broadcast_to(lax.rsqrt(deg), (_BMR, H))


def _round_body(a0_ref, a1_ref, z_ref, dis_ref, w_ref, b_ref, zo_ref):
    dis = dis_ref[...]
    prop = dis * (a0_ref[0, :, :H] + a1_ref[0, :, :H] + z_ref[:, :H])
    h = jnp.maximum(
        jnp.dot(prop, w_ref[...], preferred_element_type=jnp.float32)
        + b_ref[...], 0.0)
    z = h * dis
    zo_ref[...] = jnp.concatenate([z, z, z, z], axis=1)


def _final_body(a0_ref, a1_ref, z_ref, dis_ref, w7_ref, b7_ref,
                w8_ref, b8_ref, o_ref):
    prop = dis_ref[...] * (a0_ref[0, :, :H] + a1_ref[0, :, :H]
                           + z_ref[:, :H])
    h7 = jnp.maximum(
        jnp.dot(prop, w7_ref[...], preferred_element_type=jnp.float32)
        + b7_ref[...], 0.0)
    o_ref[...] = jnp.dot(h7, w8_ref[...],
                         preferred_element_type=jnp.float32) + b8_ref[...]


_BMR = 2560
_GRIDR = NPAD // _BMR
_BM = 2000
_GRIDF = N // _BM


def _rspec(rows, width):
    return pl.BlockSpec((rows, width), lambda i: (i, 0))


def _sspec(rows, c):
    return pl.BlockSpec((1, rows, 128), lambda i: (c, i, 0))


def _full_spec(shape):
    return pl.BlockSpec(shape, lambda i: (0,) * len(shape))


_params = pltpu.CompilerParams(dimension_semantics=("arbitrary",))

_mm0 = pl.pallas_call(
    _mm0_body,
    grid=(_GRIDR,),
    in_specs=[_rspec(_BMR, DIN), _full_spec((DIN, H)), _full_spec((1, H)),
              _rspec(_BMR, H)],
    out_specs=_rspec(_BMR, 128),
    out_shape=jax.ShapeDtypeStruct((NPAD, 128), jnp.float32),
    compiler_params=_params,
)

_dis_call = pl.pallas_call(
    _dis_body,
    grid=(_GRIDR,),
    in_specs=[_sspec(_BMR, 0), _sspec(_BMR, 1)],
    out_specs=_rspec(_BMR, H),
    out_shape=jax.ShapeDtypeStruct((NPAD, H), jnp.float32),
    compiler_params=_params,
)

_round = pl.pallas_call(
    _round_body,
    grid=(_GRIDR,),
    in_specs=[_sspec(_BMR, 0), _sspec(_BMR, 1), _rspec(_BMR, 128),
              _rspec(_BMR, H),
              _full_spec((H, H)), _full_spec((1, H))],
    out_specs=_rspec(_BMR, 128),
    out_shape=jax.ShapeDtypeStruct((NPAD, 128), jnp.float32),
    compiler_params=_params,
)

_final = pl.pallas_call(
    _final_body,
    grid=(_GRIDF,),
    in_specs=[_sspec(_BM, 0), _sspec(_BM, 1), _rspec(_BM, 128),
              _rspec(_BM, H),
              _full_spec((H, H)), _full_spec((1, H)),
              _full_spec((H, DOUT)), _full_spec((1, DOUT))],
    out_specs=_rspec(_BM, DOUT),
    out_shape=jax.ShapeDtypeStruct((N, DOUT), jnp.float32),
    compiler_params=_params,
)


def kernel(x, edge_index, W0, b0, W1, b1, W2, b2, W3, b3, W4, b4, W5, b5,
           W6, b6, W7, b7, W8, b8):
    src = edge_index[0]
    dst = edge_index[1]
    pad = EP - E
    pad_src = jnp.arange(pad, dtype=jnp.int32) % N
    pad_dst = N + jnp.arange(pad, dtype=jnp.int32) % (NPAD - N)
    srcp = (4 * jnp.concatenate([src, pad_src])).reshape(NW, NCHUNK, CHUNK)
    dstp = jnp.concatenate([dst, pad_dst]).reshape(NW, NCHUNK, CHUNK)
    zeros_h = jnp.zeros((NPAD, H), jnp.float32)
    zeros_8 = jnp.zeros((NPAD, 8), jnp.float32)
    ones_8 = jnp.ones((CHUNK, 8), jnp.float32)
    x_pad = jnp.concatenate(
        [x, jnp.zeros((NPAD - N, DIN), jnp.float32)], axis=0)

    degp = _deg_call(dstp, ones_8, zeros_8)
    dis = _dis_call(degp, degp)

    z = _mm0(x_pad, W0, b0.reshape(1, H), dis)
    for Wi, bi in ((W1, b1), (W2, b2), (W3, b3), (W4, b4), (W5, b5),
                   (W6, b6)):
        acc = _prop_call(z.reshape(4 * NPAD, H), srcp, dstp, zeros_h)
        z = _round(acc, acc, z, dis, Wi, bi.reshape(1, H))
    acc = _prop_call(z.reshape(4 * NPAD, H), srcp, dstp, zeros_h)
    return _final(acc, acc, z, dis, W7, b7.reshape(1, H),
                  W8, b8.reshape(1, DOUT))
